# baseline (speedup 1.0000x reference)
import functools

import jax
import jax.numpy as jnp
from jax import lax
from jax.experimental import pallas as pl
from jax.experimental.pallas import tpu as pltpu
from jax.experimental.pallas import tpu_sc as plsc

HIDDEN = 1024
FFN = 4096
E = 8
TOKENS = 2048

FT = 512
NF = FFN // FT
TCH = 256
MAXCH = TOKENS // TCH
NSLOT = 6144
NW = 16
APW = 256
ZW = NSLOT // NW
NWORK = 32
GCH = 64
GPW = NSLOT // NWORK // GCH
CCH = 16
CPW = TOKENS // NWORK // CCH

_mesh = plsc.VectorSubcoreMesh(core_axis_name="c", subcore_axis_name="s")


def _router_body(x_ref, gw_ref, logits_ref, e1_ref, e2_ref, w1_ref, w2_ref):
    xf = x_ref[...]
    gw = gw_ref[...]
    logits = jnp.dot(xf, gw, preferred_element_type=jnp.float32)
    logits_ref[...] = logits
    rw = jax.nn.softmax(logits, axis=-1)
    ids = jax.lax.broadcasted_iota(jnp.int32, rw.shape, 1)
    m1 = jnp.max(rw, axis=-1, keepdims=True)
    e1 = jnp.min(jnp.where(rw == m1, ids, E), axis=-1, keepdims=True)
    rw2 = jnp.where(ids == e1, -jnp.inf, rw)
    m2 = jnp.max(rw2, axis=-1, keepdims=True)
    e2 = jnp.min(jnp.where(rw2 == m2, ids, E), axis=-1, keepdims=True)
    s = m1 + m2
    e1_ref[...] = e1
    e2_ref[...] = e2
    w1_ref[...] = m1 / s
    w2_ref[...] = m2 / s


@functools.partial(
    pl.kernel,
    mesh=_mesh,
    out_type=(
        jax.ShapeDtypeStruct((NSLOT,), jnp.int32),
        jax.ShapeDtypeStruct((NSLOT,), jnp.float32),
        jax.ShapeDtypeStruct((TOKENS,), jnp.int32),
        jax.ShapeDtypeStruct((TOKENS,), jnp.int32),
        jax.ShapeDtypeStruct((128,), jnp.int32),
        jax.ShapeDtypeStruct((128,), jnp.int32),
    ),
    scratch_types=[
        pltpu.VMEM((APW,), jnp.int32),
        pltpu.VMEM((APW,), jnp.float32),
        pltpu.VMEM((APW,), jnp.int32),
        pltpu.VMEM((2, 128), jnp.int32),
        pltpu.VMEM((128,), jnp.int32),
        pltpu.VMEM((NW * 128,), jnp.int32),
        pltpu.VMEM((ZW,), jnp.int32),
        pltpu.VMEM((ZW,), jnp.float32),
        pltpu.VMEM((NSLOT,), jnp.int32),
        pltpu.VMEM((NSLOT,), jnp.float32),
        pltpu.VMEM((128,), jnp.int32),
        pltpu.VMEM((128,), jnp.int32),
        pltpu.VMEM_SHARED((NW * 128,), jnp.int32),
        pltpu.VMEM_SHARED((NSLOT,), jnp.int32),
        pltpu.VMEM_SHARED((NSLOT,), jnp.float32),
        pltpu.SMEM((16,), jnp.int32),
        pltpu.SMEM((16,), jnp.int32),
        pltpu.SMEM((APW,), jnp.int32),
        pltpu.SMEM((16,), jnp.int32),
        pltpu.SMEM((16,), jnp.int32),
        pltpu.VMEM((APW,), jnp.int32),
    ],
)
def _dispatch(e1_hbm, e2_hbm, w1_hbm, w2_hbm,
              tok_hbm, ws_hbm, inv1_hbm, inv2_hbm, base_hbm, nch_hbm,
              ev_v, wv_v, tokv_v, slot_v, lcnt_v, allc_v,
              zb_i, zb_f, tmp_tok, tmp_ws, basev, nchv,
              sh_cnt, sh_tok, sh_ws, cnt_s, start_s, rloc_s,
              tot_s, pre_s, slot_flat):
    c = lax.axis_index("c")
    w = lax.axis_index("s")

    @pl.when(c == 0)
    def _core0():
        @pl.when(w < 8)
        def _load1():
            pltpu.sync_copy(e1_hbm.at[pl.ds(w * APW, APW)], ev_v)
            pltpu.sync_copy(w1_hbm.at[pl.ds(w * APW, APW)], wv_v)

        @pl.when(w >= 8)
        def _load2():
            pltpu.sync_copy(e2_hbm.at[pl.ds((w - 8) * APW, APW)], ev_v)
            pltpu.sync_copy(w2_hbm.at[pl.ds((w - 8) * APW, APW)], wv_v)

        lane = lax.iota(jnp.int32, 16)
        for k in range(16):
            cnt_s[k] = 0
        zi = jnp.zeros((16,), jnp.int32)
        zf = jnp.zeros((16,), jnp.float32)
        for k in range(ZW // 16):
            zb_i[pl.ds(k * 16, 16)] = zi
            zb_f[pl.ds(k * 16, 16)] = zf
        tbase = lax.rem(w, 8) * APW
        for k in range(APW // 16):
            tokv_v[pl.ds(k * 16, 16)] = tbase + k * 16 + lane

        @pl.loop(0, APW // 16)
        def _p1(j):
            ev16 = ev_v[pl.ds(j * 16, 16)]
            for i in range(16):
                e = ev16[i]
                r = cnt_s[e]
                cnt_s[e] = r + 1
                rloc_s[j * 16 + i] = r

        lv = jnp.zeros((16,), jnp.int32)
        for k in range(8):
            lv = jnp.where(lane == k, cnt_s[k], lv)
        lcnt_v[pl.ds(0, 16)] = lv
        for k in range(1, 8):
            lcnt_v[pl.ds(k * 16, 16)] = zi
        pltpu.sync_copy(lcnt_v, sh_cnt.at[pl.ds(w * 128, 128)])
        plsc.subcore_barrier()
        pltpu.sync_copy(sh_cnt, allc_v)

        for e in range(8):
            tot_s[e] = 0
            pre_s[e] = 0

        @pl.loop(0, NW)
        def _p2(w2):
            av = allc_v[pl.ds(w2 * 128, 16)]
            for e in range(8):
                v = av[e]
                tot_s[e] = tot_s[e] + v
                pre_s[e] = pre_s[e] + jnp.where(w2 < w, v, 0)

        base_e = jnp.int32(0)
        bv = jnp.zeros((16,), jnp.int32)
        nv = jnp.zeros((16,), jnp.int32)
        for e in range(8):
            nch_e = (tot_s[e] + (TCH - 1)) // TCH
            start_s[e] = base_e + pre_s[e]
            bv = jnp.where(lane == e, base_e, bv)
            nv = jnp.where(lane == e, nch_e, nv)
            base_e = base_e + nch_e * TCH
        basev[pl.ds(0, 16)] = bv
        nchv[pl.ds(0, 16)] = nv
        for k in range(1, 8):
            basev[pl.ds(k * 16, 16)] = zi
            nchv[pl.ds(k * 16, 16)] = zi

        pltpu.sync_copy(zb_i, sh_tok.at[pl.ds(w * ZW, ZW)])
        pltpu.sync_copy(zb_f, sh_ws.at[pl.ds(w * ZW, ZW)])
        plsc.subcore_barrier()

        @pl.loop(0, APW // 16)
        def _p3(j):
            ev16 = ev_v[pl.ds(j * 16, 16)]
            sv = jnp.zeros((16,), jnp.int32)
            for i in range(16):
                e = ev16[i]
                sv = jnp.where(lane == i, start_s[e] + rloc_s[j * 16 + i], sv)
            slot_flat[pl.ds(j * 16, 16)] = sv

        for k in range(APW // 16):
            slot_v[k // 8, pl.ds((k % 8) * 16, 16)] = slot_flat[pl.ds(k * 16, 16)]
        for j in range(2):
            pltpu.sync_copy(tokv_v.at[pl.ds(j * 128, 128)],
                            sh_tok.at[slot_v.at[j]])
            pltpu.sync_copy(wv_v.at[pl.ds(j * 128, 128)],
                            sh_ws.at[slot_v.at[j]])

        @pl.when(w < 8)
        def _inv1():
            for j in range(2):
                pltpu.sync_copy(slot_v.at[j],
                                inv1_hbm.at[pl.ds(w * APW + j * 128, 128)])

        @pl.when(w >= 8)
        def _inv2():
            for j in range(2):
                pltpu.sync_copy(
                    slot_v.at[j],
                    inv2_hbm.at[pl.ds((w - 8) * APW + j * 128, 128)])

        plsc.subcore_barrier()

        @pl.when(w == 0)
        def _writeout():
            pltpu.sync_copy(sh_tok, tmp_tok)
            pltpu.sync_copy(tmp_tok, tok_hbm)
            pltpu.sync_copy(sh_ws, tmp_ws)
            pltpu.sync_copy(tmp_ws, ws_hbm)
            pltpu.sync_copy(basev, base_hbm)
            pltpu.sync_copy(nchv, nch_hbm)


@functools.partial(
    pl.kernel,
    mesh=_mesh,
    out_type=jax.ShapeDtypeStruct((NSLOT, HIDDEN // 2), jnp.float32),
    scratch_types=[
        pltpu.VMEM((GCH,), jnp.int32),
        pltpu.VMEM((GCH, HIDDEN // 2), jnp.float32),
        pltpu.SemaphoreType.DMA,
    ],
)
def _gather(xb_hbm, tok_hbm, xs_hbm, idx_v, rows_v, sem):
    wid = lax.axis_index("s") * 2 + lax.axis_index("c")
    for k in range(GPW):
        base = wid * (GPW * GCH) + k * GCH
        pltpu.sync_copy(tok_hbm.at[pl.ds(base, GCH)], idx_v)
        pltpu.async_copy(xb_hbm.at[idx_v], rows_v, sem).wait()
        pltpu.sync_copy(rows_v, xs_hbm.at[pl.ds(base, GCH)])


@functools.partial(
    pl.kernel,
    mesh=_mesh,
    out_type=jax.ShapeDtypeStruct((TOKENS, HIDDEN), jnp.float32),
    scratch_types=[
        pltpu.VMEM((CCH,), jnp.int32),
        pltpu.VMEM((CCH,), jnp.int32),
        pltpu.VMEM((CCH, HIDDEN), jnp.float32),
        pltpu.VMEM((CCH, HIDDEN), jnp.float32),
        pltpu.SemaphoreType.DMA,
    ],
)
def _combine(ys_hbm, inv1_hbm, inv2_hbm, out_hbm, i1_v, i2_v, t1, t2, sem):
    wid = lax.axis_index("s") * 2 + lax.axis_index("c")

    @pl.loop(0, CPW)
    def _step(k):
        b = wid * (CPW * CCH) + k * CCH
        pltpu.sync_copy(inv1_hbm.at[pl.ds(b, CCH)], i1_v)
        pltpu.sync_copy(inv2_hbm.at[pl.ds(b, CCH)], i2_v)
        pltpu.async_copy(ys_hbm.at[i1_v], t1, sem).wait()
        pltpu.async_copy(ys_hbm.at[i2_v], t2, sem).wait()
        for r in range(CCH):
            for cc in range(HIDDEN // 16):
                sl = (r, pl.ds(cc * 16, 16))
                t1[sl] = t1[sl] + t2[sl]
        pltpu.sync_copy(t1, out_hbm.at[pl.ds(b, CCH)])


def _ffn_body(base_ref, nch_ref, xs_ref, ws_ref, wg_ref, wu_ref, wd_ref,
              ys_ref):
    e = pl.program_id(0)
    f = pl.program_id(1)
    base_e = base_ref[e]
    nch_e = nch_ref[e]
    wg = wg_ref[0].astype(jnp.bfloat16)
    wu = wu_ref[0].astype(jnp.bfloat16)
    wd = wd_ref[0].astype(jnp.bfloat16)
    for j in range(MAXCH):
        @pl.when(j < nch_e)
        def _chunk(j=j):
            row = pl.multiple_of(base_e + j * TCH, TCH)
            xt = xs_ref[pl.ds(row, TCH), :]
            g = jnp.dot(xt, wg, preferred_element_type=jnp.float32)
            u = jnp.dot(xt, wu, preferred_element_type=jnp.float32)
            h = (g * jax.nn.sigmoid(g)) * u
            w_col = ws_ref[pl.ds(row, TCH), :]
            hb = (h * w_col).astype(jnp.bfloat16)
            acc = jnp.dot(hb, wd, preferred_element_type=jnp.float32)

            @pl.when(f == 0)
            def _w():
                ys_ref[pl.ds(row, TCH), :] = acc

            @pl.when(f > 0)
            def _a():
                ys_ref[pl.ds(row, TCH), :] += acc


def kernel(x, gate_w, w_gate, w_up, w_down):
    B, S, D = x.shape
    xf = x.reshape(S, D)

    logits, e1, e2, w1, w2 = pl.pallas_call(
        _router_body,
        out_shape=(
            jax.ShapeDtypeStruct((TOKENS, E), jnp.float32),
            jax.ShapeDtypeStruct((TOKENS, 1), jnp.int32),
            jax.ShapeDtypeStruct((TOKENS, 1), jnp.int32),
            jax.ShapeDtypeStruct((TOKENS, 1), jnp.float32),
            jax.ShapeDtypeStruct((TOKENS, 1), jnp.float32),
        ),
    )(xf, gate_w)

    tok, ws, inv1, inv2, base, nch = _dispatch(
        e1.reshape(TOKENS), e2.reshape(TOKENS),
        w1.reshape(TOKENS), w2.reshape(TOKENS))
    base = base[:8]
    nch = nch[:8]

    xbf = xf.astype(jnp.bfloat16)
    xb32 = lax.bitcast_convert_type(
        xbf.reshape(TOKENS, HIDDEN // 2, 2), jnp.float32)
    xs32 = _gather(xb32, tok)
    xs_bf = lax.bitcast_convert_type(xs32, jnp.bfloat16).reshape(NSLOT, HIDDEN)

    ys = pl.pallas_call(
        _ffn_body,
        grid_spec=pltpu.PrefetchScalarGridSpec(
            num_scalar_prefetch=2,
            grid=(E, NF),
            in_specs=[
                pl.BlockSpec((NSLOT, HIDDEN), lambda e, f, *_: (0, 0)),
                pl.BlockSpec((NSLOT, 1), lambda e, f, *_: (0, 0)),
                pl.BlockSpec((1, HIDDEN, FT), lambda e, f, *_: (e, 0, f)),
                pl.BlockSpec((1, HIDDEN, FT), lambda e, f, *_: (e, 0, f)),
                pl.BlockSpec((1, FT, HIDDEN), lambda e, f, *_: (e, f, 0)),
            ],
            out_specs=pl.BlockSpec((NSLOT, HIDDEN), lambda e, f, *_: (0, 0)),
        ),
        out_shape=jax.ShapeDtypeStruct((NSLOT, HIDDEN), jnp.float32),
    )(base, nch, xs_bf, ws.reshape(NSLOT, 1), w_gate, w_up, w_down)

    out = _combine(ys, inv1, inv2)
    return out.reshape(B, S, D), logits


# speedup vs baseline: 1.0276x; 1.0276x over previous
import dataclasses
import functools

import jax
import jax.numpy as jnp
from jax import lax
from jax.experimental import pallas as pl
from jax.experimental.pallas import tpu as pltpu
from jax.experimental.pallas import tpu_sc as plsc

HIDDEN = 1024
FFN = 4096
E = 8
TOKENS = 2048

FT = 512
NF = FFN // FT
TCH = 256
MAXCH = TOKENS // TCH
NSLOT = 6144
NW = 16
APW = 256
ZW = NSLOT // NW
NWORK = 32
GCH = 64
GPW = NSLOT // NWORK // GCH
CCH = 16
TPW = TOKENS // NWORK
CPW = TPW // CCH

_mesh = plsc.VectorSubcoreMesh(core_axis_name="c", subcore_axis_name="s")

_sc_params = pltpu.CompilerParams()
if "needs_layout_passes" in pltpu.CompilerParams.__dataclass_fields__:
    _sc_params = dataclasses.replace(_sc_params, needs_layout_passes=False)


def _router_body(x_ref, gw_ref, logits_ref, e1_ref, e2_ref, w1_ref, w2_ref):
    xf = x_ref[...]
    gw = gw_ref[...]
    logits = jnp.dot(xf, gw, preferred_element_type=jnp.float32)
    logits_ref[...] = logits
    rw = jax.nn.softmax(logits, axis=-1)
    ids = jax.lax.broadcasted_iota(jnp.int32, rw.shape, 1)
    m1 = jnp.max(rw, axis=-1, keepdims=True)
    e1 = jnp.min(jnp.where(rw == m1, ids, E), axis=-1, keepdims=True)
    rw2 = jnp.where(ids == e1, -jnp.inf, rw)
    m2 = jnp.max(rw2, axis=-1, keepdims=True)
    e2 = jnp.min(jnp.where(rw2 == m2, ids, E), axis=-1, keepdims=True)
    s = m1 + m2
    e1_ref[...] = e1
    e2_ref[...] = e2
    w1_ref[...] = m1 / s
    w2_ref[...] = m2 / s


@functools.partial(
    pl.kernel,
    mesh=_mesh,
    compiler_params=_sc_params,
    out_type=(
        jax.ShapeDtypeStruct((NSLOT,), jnp.int32),
        jax.ShapeDtypeStruct((NSLOT,), jnp.float32),
        jax.ShapeDtypeStruct((TOKENS,), jnp.int32),
        jax.ShapeDtypeStruct((TOKENS,), jnp.int32),
        jax.ShapeDtypeStruct((128,), jnp.int32),
        jax.ShapeDtypeStruct((128,), jnp.int32),
    ),
    scratch_types=[
        pltpu.VMEM((APW,), jnp.int32),
        pltpu.VMEM((APW,), jnp.float32),
        pltpu.VMEM((APW,), jnp.int32),
        pltpu.VMEM((2, 128), jnp.int32),
        pltpu.VMEM((128,), jnp.int32),
        pltpu.VMEM((NW * 128,), jnp.int32),
        pltpu.VMEM((ZW,), jnp.int32),
        pltpu.VMEM((ZW,), jnp.float32),
        pltpu.VMEM((NSLOT,), jnp.int32),
        pltpu.VMEM((NSLOT,), jnp.float32),
        pltpu.VMEM((128,), jnp.int32),
        pltpu.VMEM((128,), jnp.int32),
        pltpu.VMEM_SHARED((NW * 128,), jnp.int32),
        pltpu.VMEM_SHARED((NSLOT,), jnp.int32),
        pltpu.VMEM_SHARED((NSLOT,), jnp.float32),
        pltpu.SMEM((16,), jnp.int32),
        pltpu.SMEM((16,), jnp.int32),
        pltpu.VMEM((APW,), jnp.int32),
        pltpu.SMEM((16,), jnp.int32),
        pltpu.SMEM((16,), jnp.int32),
        pltpu.VMEM((APW,), jnp.int32),
        pltpu.VMEM((16,), jnp.int32),
    ],
)
def _dispatch(e1_hbm, e2_hbm, w1_hbm, w2_hbm,
              tok_hbm, ws_hbm, inv1_hbm, inv2_hbm, base_hbm, nch_hbm,
              ev_v, wv_v, tokv_v, slot_v, lcnt_v, allc_v,
              zb_i, zb_f, tmp_tok, tmp_ws, basev, nchv,
              sh_cnt, sh_tok, sh_ws, cnt_s, start_s, rloc_v,
              tot_s, pre_s, slot_flat, start_v):
    c = lax.axis_index("c")
    w = lax.axis_index("s")

    @pl.when(c == 0)
    def _core0():
        @pl.when(w < 8)
        def _load1():
            pltpu.sync_copy(e1_hbm.at[pl.ds(w * APW, APW)], ev_v)
            pltpu.sync_copy(w1_hbm.at[pl.ds(w * APW, APW)], wv_v)

        @pl.when(w >= 8)
        def _load2():
            pltpu.sync_copy(e2_hbm.at[pl.ds((w - 8) * APW, APW)], ev_v)
            pltpu.sync_copy(w2_hbm.at[pl.ds((w - 8) * APW, APW)], wv_v)

        lane = lax.iota(jnp.int32, 16)
        for k in range(16):
            cnt_s[k] = 0
        zi = jnp.zeros((16,), jnp.int32)
        zf = jnp.zeros((16,), jnp.float32)
        for k in range(ZW // 16):
            zb_i[pl.ds(k * 16, 16)] = zi
            zb_f[pl.ds(k * 16, 16)] = zf
        tbase = lax.rem(w, 8) * APW
        for k in range(APW // 16):
            tokv_v[pl.ds(k * 16, 16)] = tbase + k * 16 + lane

        @pl.loop(0, APW // 16)
        def _p1(j):
            ev16 = ev_v[pl.ds(j * 16, 16)]
            r16 = jnp.zeros((16,), jnp.int32)
            for e in range(8):
                m = ev16 == e
                cs = plsc.cumsum(jnp.where(m, 1, 0))
                prior = cnt_s[e]
                r16 = jnp.where(m, prior + cs - 1, r16)
                cnt_s[e] = prior + cs[15]
            rloc_v[pl.ds(j * 16, 16)] = r16

        lv = jnp.zeros((16,), jnp.int32)
        for k in range(8):
            lv = jnp.where(lane == k, cnt_s[k], lv)
        lcnt_v[pl.ds(0, 16)] = lv
        for k in range(1, 8):
            lcnt_v[pl.ds(k * 16, 16)] = zi
        pltpu.sync_copy(lcnt_v, sh_cnt.at[pl.ds(w * 128, 128)])
        plsc.subcore_barrier()
        pltpu.sync_copy(sh_cnt, allc_v)

        for e in range(8):
            tot_s[e] = 0
            pre_s[e] = 0

        @pl.loop(0, NW)
        def _p2(w2):
            av = allc_v[pl.ds(w2 * 128, 16)]
            for e in range(8):
                v = av[e]
                tot_s[e] = tot_s[e] + v
                pre_s[e] = pre_s[e] + jnp.where(w2 < w, v, 0)

        base_e = jnp.int32(0)
        bv = jnp.zeros((16,), jnp.int32)
        nv = jnp.zeros((16,), jnp.int32)
        for e in range(8):
            nch_e = (tot_s[e] + (TCH - 1)) // TCH
            start_s[e] = base_e + pre_s[e]
            bv = jnp.where(lane == e, base_e, bv)
            nv = jnp.where(lane == e, nch_e, nv)
            base_e = base_e + nch_e * TCH
        basev[pl.ds(0, 16)] = bv
        nchv[pl.ds(0, 16)] = nv
        for k in range(1, 8):
            basev[pl.ds(k * 16, 16)] = zi
            nchv[pl.ds(k * 16, 16)] = zi

        pltpu.sync_copy(zb_i, sh_tok.at[pl.ds(w * ZW, ZW)])
        pltpu.sync_copy(zb_f, sh_ws.at[pl.ds(w * ZW, ZW)])
        plsc.subcore_barrier()

        sv16 = jnp.zeros((16,), jnp.int32)
        for e in range(8):
            sv16 = jnp.where(lane == e, start_s[e], sv16)
        start_v[...] = sv16

        @pl.loop(0, APW // 16)
        def _p3(j):
            ev16 = ev_v[pl.ds(j * 16, 16)]
            sg = plsc.load_gather(start_v, [ev16])
            slot_flat[pl.ds(j * 16, 16)] = sg + rloc_v[pl.ds(j * 16, 16)]

        for k in range(APW // 16):
            slot_v[k // 8, pl.ds((k % 8) * 16, 16)] = slot_flat[pl.ds(k * 16, 16)]
        for j in range(2):
            pltpu.sync_copy(tokv_v.at[pl.ds(j * 128, 128)],
                            sh_tok.at[slot_v.at[j]])
            pltpu.sync_copy(wv_v.at[pl.ds(j * 128, 128)],
                            sh_ws.at[slot_v.at[j]])

        @pl.when(w < 8)
        def _inv1():
            for j in range(2):
                pltpu.sync_copy(slot_v.at[j],
                                inv1_hbm.at[pl.ds(w * APW + j * 128, 128)])

        @pl.when(w >= 8)
        def _inv2():
            for j in range(2):
                pltpu.sync_copy(
                    slot_v.at[j],
                    inv2_hbm.at[pl.ds((w - 8) * APW + j * 128, 128)])

        plsc.subcore_barrier()

        @pl.when(w == 0)
        def _writeout():
            pltpu.sync_copy(sh_tok, tmp_tok)
            pltpu.sync_copy(tmp_tok, tok_hbm)
            pltpu.sync_copy(sh_ws, tmp_ws)
            pltpu.sync_copy(tmp_ws, ws_hbm)
            pltpu.sync_copy(basev, base_hbm)
            pltpu.sync_copy(nchv, nch_hbm)


@functools.partial(
    pl.kernel,
    mesh=_mesh,
    out_type=jax.ShapeDtypeStruct((NSLOT, HIDDEN // 2), jnp.float32),
    scratch_types=[
        pltpu.VMEM((GCH,), jnp.int32),
        pltpu.VMEM((GCH, HIDDEN // 2), jnp.float32),
        pltpu.SemaphoreType.DMA,
    ],
)
def _gather(xb_hbm, tok_hbm, xs_hbm, idx_v, rows_v, sem):
    wid = lax.axis_index("s") * 2 + lax.axis_index("c")
    for k in range(GPW):
        base = wid * (GPW * GCH) + k * GCH
        pltpu.sync_copy(tok_hbm.at[pl.ds(base, GCH)], idx_v)
        pltpu.async_copy(xb_hbm.at[idx_v], rows_v, sem).wait()
        pltpu.sync_copy(rows_v, xs_hbm.at[pl.ds(base, GCH)])


@functools.partial(
    pl.kernel,
    mesh=_mesh,
    out_type=jax.ShapeDtypeStruct((2 * TOKENS, HIDDEN), jnp.float32),
    scratch_types=[
        pltpu.VMEM((TPW,), jnp.int32),
        pltpu.VMEM((TPW,), jnp.int32),
        pltpu.VMEM((2, CCH, HIDDEN), jnp.float32),
        pltpu.VMEM((2, CCH, HIDDEN), jnp.float32),
        pltpu.SemaphoreType.DMA,
        pltpu.SemaphoreType.DMA,
        pltpu.SemaphoreType.DMA,
        pltpu.SemaphoreType.DMA,
    ],
)
def _combine(ys_hbm, inv1_hbm, inv2_hbm, out_hbm,
             i1all, i2all, t1, t2, sg1, sg2, sw1, sw2):
    wid = lax.axis_index("s") * 2 + lax.axis_index("c")
    tb = wid * TPW
    pltpu.sync_copy(inv1_hbm.at[pl.ds(tb, TPW)], i1all)
    pltpu.sync_copy(inv2_hbm.at[pl.ds(tb, TPW)], i2all)
    wr = []
    for k in range(CPW):
        b = k % 2
        if k >= 2:
            wr[2 * (k - 2)].wait()
            wr[2 * (k - 2) + 1].wait()
        g1 = pltpu.async_copy(ys_hbm.at[i1all.at[pl.ds(k * CCH, CCH)]],
                              t1.at[b], sg1)
        g2 = pltpu.async_copy(ys_hbm.at[i2all.at[pl.ds(k * CCH, CCH)]],
                              t2.at[b], sg2)
        g1.wait()
        g2.wait()
        wr.append(pltpu.async_copy(
            t1.at[b], out_hbm.at[pl.ds(tb + k * CCH, CCH)], sw1))
        wr.append(pltpu.async_copy(
            t2.at[b], out_hbm.at[pl.ds(TOKENS + tb + k * CCH, CCH)], sw2))
    for d in wr[2 * max(0, CPW - 2):]:
        d.wait()


def _add_body(a_ref, b_ref, o_ref):
    o_ref[...] = a_ref[...] + b_ref[...]


def _ffn_body(base_ref, nch_ref, xs_ref, ws_ref, wg_ref, wu_ref, wd_ref,
              ys_ref):
    e = pl.program_id(0)
    f = pl.program_id(1)
    base_e = base_ref[e]
    nch_e = nch_ref[e]
    wg = wg_ref[0].astype(jnp.bfloat16)
    wu = wu_ref[0].astype(jnp.bfloat16)
    wd = wd_ref[0].astype(jnp.bfloat16)
    for j in range(MAXCH):
        @pl.when(j < nch_e)
        def _chunk(j=j):
            row = pl.multiple_of(base_e + j * TCH, TCH)
            xt = xs_ref[pl.ds(row, TCH), :]
            g = jnp.dot(xt, wg, preferred_element_type=jnp.float32)
            u = jnp.dot(xt, wu, preferred_element_type=jnp.float32)
            h = (g * jax.nn.sigmoid(g)) * u
            w_col = ws_ref[pl.ds(row, TCH), :]
            hb = (h * w_col).astype(jnp.bfloat16)
            acc = jnp.dot(hb, wd, preferred_element_type=jnp.float32)

            @pl.when(f == 0)
            def _w():
                ys_ref[pl.ds(row, TCH), :] = acc

            @pl.when(f > 0)
            def _a():
                ys_ref[pl.ds(row, TCH), :] += acc


def kernel(x, gate_w, w_gate, w_up, w_down):
    B, S, D = x.shape
    xf = x.reshape(S, D)

    logits, e1, e2, w1, w2 = pl.pallas_call(
        _router_body,
        out_shape=(
            jax.ShapeDtypeStruct((TOKENS, E), jnp.float32),
            jax.ShapeDtypeStruct((TOKENS, 1), jnp.int32),
            jax.ShapeDtypeStruct((TOKENS, 1), jnp.int32),
            jax.ShapeDtypeStruct((TOKENS, 1), jnp.float32),
            jax.ShapeDtypeStruct((TOKENS, 1), jnp.float32),
        ),
    )(xf, gate_w)

    tok, ws, inv1, inv2, base, nch = _dispatch(
        e1.reshape(TOKENS), e2.reshape(TOKENS),
        w1.reshape(TOKENS), w2.reshape(TOKENS))
    base = base[:8]
    nch = nch[:8]

    xbf = xf.astype(jnp.bfloat16)
    xb32 = lax.bitcast_convert_type(
        xbf.reshape(TOKENS, HIDDEN // 2, 2), jnp.float32)
    xs32 = _gather(xb32, tok)
    xs_bf = lax.bitcast_convert_type(xs32, jnp.bfloat16).reshape(NSLOT, HIDDEN)

    ys = pl.pallas_call(
        _ffn_body,
        grid_spec=pltpu.PrefetchScalarGridSpec(
            num_scalar_prefetch=2,
            grid=(E, NF),
            in_specs=[
                pl.BlockSpec((NSLOT, HIDDEN), lambda e, f, *_: (0, 0)),
                pl.BlockSpec((NSLOT, 1), lambda e, f, *_: (0, 0)),
                pl.BlockSpec((1, HIDDEN, FT), lambda e, f, *_: (e, 0, f)),
                pl.BlockSpec((1, HIDDEN, FT), lambda e, f, *_: (e, 0, f)),
                pl.BlockSpec((1, FT, HIDDEN), lambda e, f, *_: (e, f, 0)),
            ],
            out_specs=pl.BlockSpec((NSLOT, HIDDEN), lambda e, f, *_: (0, 0)),
        ),
        out_shape=jax.ShapeDtypeStruct((NSLOT, HIDDEN), jnp.float32),
    )(base, nch, xs_bf, ws.reshape(NSLOT, 1), w_gate, w_up, w_down)

    cat = _combine(ys, inv1, inv2)
    out = pl.pallas_call(
        _add_body,
        grid=(8,),
        in_specs=[pl.BlockSpec((TOKENS // 8, HIDDEN), lambda i: (i, 0)),
                  pl.BlockSpec((TOKENS // 8, HIDDEN), lambda i: (i + 8, 0))],
        out_specs=pl.BlockSpec((TOKENS // 8, HIDDEN), lambda i: (i, 0)),
        out_shape=jax.ShapeDtypeStruct((TOKENS, HIDDEN), jnp.float32),
    )(cat, cat)
    return out.reshape(B, S, D), logits
